# Initial kernel scaffold; baseline (speedup 1.0000x reference)
import functools

import jax
import jax.numpy as jnp
from jax import lax
from jax.experimental import pallas as pl
from jax.experimental.pallas import tpu as pltpu
from jax.experimental.pallas import tpu_sc as plsc

_D = 128
_B = 4096
_S = 50
_LE = 20
_NW = 32
_CHUNK = 128

_NTEXT = _B * _S
_TCHUNKS = _NTEXT // _CHUNK
_TCH_W = _TCHUNKS // _NW
_EB_W = _B // _NW
_NROWS = _NTEXT + _B


def _sc_gather(table, tidx2d, eidx3d):
    mesh = plsc.VectorSubcoreMesh(core_axis_name="c", subcore_axis_name="s")

    @functools.partial(
        pl.kernel,
        mesh=mesh,
        out_type=jax.ShapeDtypeStruct((_NROWS, _D), jnp.float32),
        scratch_types=[
            pltpu.VMEM((_TCH_W, _CHUNK), jnp.int32),
            pltpu.VMEM((_LE, _CHUNK), jnp.int32),
            pltpu.VMEM((_CHUNK, _D), jnp.float32),
            pltpu.VMEM((_EB_W, _D), jnp.float32),
            pltpu.SemaphoreType.DMA,
        ],
    )
    def gather_kernel(table_hbm, tidx_hbm, eidx_hbm, out_hbm,
                      idx_v, eidx_v, rows_v, acc_v, sem):
        wid = lax.axis_index("s") * 2 + lax.axis_index("c")

        pltpu.sync_copy(tidx_hbm.at[pl.ds(wid * _TCH_W, _TCH_W)], idx_v)

        def tbody(j, carry):
            pltpu.async_copy(table_hbm.at[idx_v.at[j]], rows_v, sem).wait()
            pltpu.sync_copy(
                rows_v, out_hbm.at[pl.ds((wid * _TCH_W + j) * _CHUNK, _CHUNK)])
            return carry

        lax.fori_loop(0, _TCH_W, tbody, 0)

        pltpu.sync_copy(eidx_hbm.at[wid], eidx_v)
        pltpu.async_copy(table_hbm.at[eidx_v.at[0]], acc_v, sem).wait()

        def ebody(l, carry):
            pltpu.async_copy(table_hbm.at[eidx_v.at[l]], acc_v, sem,
                             add=True).wait()
            return carry

        lax.fori_loop(1, _LE, ebody, 0)
        pltpu.sync_copy(acc_v, out_hbm.at[pl.ds(_NTEXT + wid * _EB_W, _EB_W)])

    return gather_kernel(table, tidx2d, eidx3d)


def _mlp_body(x_ref, w1_ref, b1_ref, w2_ref, b2_ref, w3_ref, b3_ref, o_ref):
    x = x_ref[...]
    h = jnp.dot(x, w1_ref[...], preferred_element_type=jnp.float32) + b1_ref[...]
    h = jnp.where(h >= 0, h, 0.01 * h)
    h = jnp.dot(h, w2_ref[...], preferred_element_type=jnp.float32) + b2_ref[...]
    h = jnp.where(h >= 0, h, 0.01 * h)
    h = jnp.dot(h, w3_ref[...], preferred_element_type=jnp.float32) + b3_ref[...]
    h = jnp.where(h >= 0, h, 0.01 * h)
    o_ref[...] = h[:, :30]


def _tc_mlp(rows, w1p, b1p, w2p, b2p, w3p, b3p):
    n = rows.shape[0]
    blk = 1024
    grid = (n // blk,)
    wspec = pl.BlockSpec((_D, _D), lambda i: (0, 0))
    bspec = pl.BlockSpec((1, _D), lambda i: (0, 0))
    return pl.pallas_call(
        _mlp_body,
        grid=grid,
        in_specs=[
            pl.BlockSpec((blk, _D), lambda i: (i, 0)),
            wspec, bspec, wspec, bspec, wspec, bspec,
        ],
        out_specs=pl.BlockSpec((blk, 30), lambda i: (i, 0)),
        out_shape=jax.ShapeDtypeStruct((n, 30), jnp.float32),
    )(rows, w1p, b1p, w2p, b2p, w3p, b3p)


def kernel(textC, emoC, tableC, W1, b1, W2, b2, W3, b3):
    textC = textC.astype(jnp.int32)
    emoC = emoC.astype(jnp.int32)

    tidx2d = textC.reshape(_TCHUNKS, _CHUNK)
    eidx3d = emoC.T.reshape(_LE, _NW, _EB_W).transpose(1, 0, 2)

    rows = _sc_gather(tableC, tidx2d, eidx3d)

    w1p = jnp.zeros((_D, _D), jnp.float32).at[:, :100].set(W1)
    b1p = jnp.zeros((1, _D), jnp.float32).at[0, :100].set(b1)
    w2p = jnp.zeros((_D, _D), jnp.float32).at[:100, :60].set(W2)
    b2p = jnp.zeros((1, _D), jnp.float32).at[0, :60].set(b2)
    w3p = jnp.zeros((_D, _D), jnp.float32).at[:60, :30].set(W3)
    b3p = jnp.zeros((1, _D), jnp.float32).at[0, :30].set(b3)

    out = _tc_mlp(rows, w1p, b1p, w2p, b2p, w3p, b3p)

    outputsC = out[:_NTEXT].reshape(_B, _S, 30)
    emo_out = out[_NTEXT:].reshape(_B, 1, 30)
    return (outputsC, emo_out)


# speedup vs baseline: 6.3154x; 6.3154x over previous
import functools

import jax
import jax.numpy as jnp
from jax import lax
from jax.experimental import pallas as pl
from jax.experimental.pallas import tpu as pltpu
from jax.experimental.pallas import tpu_sc as plsc

_D = 128
_B = 4096
_S = 50
_LE = 20
_NW = 32
_CHUNK = 128

_NTEXT = _B * _S
_TCHUNKS = _NTEXT // _CHUNK
_TCH_W = _TCHUNKS // _NW
_EB_W = _B // _NW
_NROWS = _NTEXT + _B


def _sc_gather(table, tidx2d, eidx3d):
    mesh = plsc.VectorSubcoreMesh(core_axis_name="c", subcore_axis_name="s")

    @functools.partial(
        pl.kernel,
        mesh=mesh,
        out_type=jax.ShapeDtypeStruct((_NROWS, _D), jnp.float32),
        scratch_types=[
            pltpu.VMEM((_TCH_W, _CHUNK), jnp.int32),

            pltpu.VMEM((_LE, _CHUNK), jnp.int32),
            pltpu.VMEM((_CHUNK, _D), jnp.float32),
            pltpu.VMEM((_EB_W, _D), jnp.float32),
            pltpu.SemaphoreType.DMA,
        ],
    )
    def gather_kernel(table_hbm, tidx_hbm, eidx_hbm, out_hbm,
                      idx_v, eidx_v, rows_v, acc_v, sem):
        wid = lax.axis_index("s") * 2 + lax.axis_index("c")

        pltpu.sync_copy(tidx_hbm.at[wid], idx_v)

        def tbody(j, carry):
            pltpu.async_copy(table_hbm.at[idx_v.at[j]], rows_v, sem).wait()
            pltpu.sync_copy(
                rows_v, out_hbm.at[pl.ds((wid * _TCH_W + j) * _CHUNK, _CHUNK)])
            return carry

        lax.fori_loop(0, _TCH_W, tbody, 0)

        pltpu.sync_copy(eidx_hbm.at[wid], eidx_v)
        pltpu.async_copy(table_hbm.at[eidx_v.at[0]], acc_v, sem).wait()

        def ebody(l, carry):
            pltpu.async_copy(table_hbm.at[eidx_v.at[l]], acc_v, sem,
                             add=True).wait()
            return carry

        lax.fori_loop(1, _LE, ebody, 0)
        pltpu.sync_copy(acc_v, out_hbm.at[pl.ds(_NTEXT + wid * _EB_W, _EB_W)])

    return gather_kernel(table, tidx2d, eidx3d)


def _mlp_body(x_ref, w1_ref, b1_ref, w2_ref, b2_ref, w3_ref, b3_ref, o_ref):
    x = x_ref[...]
    h = jnp.dot(x, w1_ref[...], preferred_element_type=jnp.float32) + b1_ref[...]
    h = jnp.where(h >= 0, h, 0.01 * h)
    h = jnp.dot(h, w2_ref[...], preferred_element_type=jnp.float32) + b2_ref[...]
    h = jnp.where(h >= 0, h, 0.01 * h)
    h = jnp.dot(h, w3_ref[...], preferred_element_type=jnp.float32) + b3_ref[...]
    h = jnp.where(h >= 0, h, 0.01 * h)
    o_ref[...] = h[:, :30]


def _tc_mlp(rows, w1p, b1p, w2p, b2p, w3p, b3p):
    n = rows.shape[0]
    blk = 1024
    grid = (n // blk,)
    wspec = pl.BlockSpec((_D, _D), lambda i: (0, 0))
    bspec = pl.BlockSpec((1, _D), lambda i: (0, 0))
    return pl.pallas_call(
        _mlp_body,
        grid=grid,
        in_specs=[
            pl.BlockSpec((blk, _D), lambda i: (i, 0)),
            wspec, bspec, wspec, bspec, wspec, bspec,
        ],
        out_specs=pl.BlockSpec((blk, 30), lambda i: (i, 0)),
        out_shape=jax.ShapeDtypeStruct((n, 30), jnp.float32),
    )(rows, w1p, b1p, w2p, b2p, w3p, b3p)


def kernel(textC, emoC, tableC, W1, b1, W2, b2, W3, b3):
    textC = textC.astype(jnp.int32)
    emoC = emoC.astype(jnp.int32)

    tidx3d = textC.reshape(_NW, _TCH_W, _CHUNK)
    eidx3d = emoC.T.reshape(_LE, _NW, _EB_W).transpose(1, 0, 2)

    rows = _sc_gather(tableC, tidx3d, eidx3d)

    w1p = jnp.zeros((_D, _D), jnp.float32).at[:, :100].set(W1)
    b1p = jnp.zeros((1, _D), jnp.float32).at[0, :100].set(b1)
    w2p = jnp.zeros((_D, _D), jnp.float32).at[:100, :60].set(W2)
    b2p = jnp.zeros((1, _D), jnp.float32).at[0, :60].set(b2)
    w3p = jnp.zeros((_D, _D), jnp.float32).at[:60, :30].set(W3)
    b3p = jnp.zeros((1, _D), jnp.float32).at[0, :30].set(b3)

    out = _tc_mlp(rows, w1p, b1p, w2p, b2p, w3p, b3p)

    outputsC = out[:_NTEXT].reshape(_B, _S, 30)
    emo_out = out[_NTEXT:].reshape(_B, 1, 30)
    return (outputsC, emo_out)
